# baseline (speedup 1.0000x reference)
import functools

import jax
import jax.numpy as jnp
from jax import lax
from jax.experimental import pallas as pl
from jax.experimental.pallas import tpu as pltpu
from jax.experimental.pallas import tpu_sc as plsc

_B, _F, _C, _R = 512, 2000, 8, 4
_T = 2048
_D = _C * _R
_L = 16
_CH = 128
_NCH = _T // _CH
_NBAD = _T - _F

_NC, _NS = 2, 16
_NW = _NC * _NS
_NBATCH = _B // _NW


def _make_packing_kernel(interpret=False):
    mesh = plsc.VectorSubcoreMesh(
        core_axis_name="c", subcore_axis_name="s",
        num_cores=_NC, num_subcores=_NS)

    @functools.partial(
        pl.kernel,
        out_type=jax.ShapeDtypeStruct((_B * _T, _D), jnp.float32),
        mesh=mesh,
        scratch_types=[
            pltpu.VMEM((_T,), jnp.int32),
            pltpu.VMEM((_T,), jnp.int32),
            pltpu.VMEM((_T,), jnp.int32),
            pltpu.VMEM((_NCH, _CH), jnp.int32),
            pltpu.VMEM((_T, _D), jnp.float32),
            pltpu.SemaphoreType.DMA,
            pltpu.SemaphoreType.DMA,
        ],
        interpret=interpret,
        compiler_params=pltpu.CompilerParams(
            needs_layout_passes=False, use_tc_tiling_on_sc=False),
    )
    def packing(ll_hbm, perm_hbm, out_hbm,
                perm_v, g_v, pinv_v, idx_v, buf_v, gsem, wsem):
        wid = lax.axis_index("s") * _NC + lax.axis_index("c")
        pltpu.sync_copy(perm_hbm, perm_v)

        lane = lax.iota(jnp.int32, _L)

        def setup_chunk(j, carry):
            pv = perm_v[pl.ds(j * _L, _L)]
            plsc.store_scatter(pinv_v, [pv], lane + j * _L)
            g_v[pl.ds(j * _L, _L)] = jnp.minimum(pv, _F - 1)
            return carry

        lax.fori_loop(0, _T // _L, setup_chunk, 0, unroll=False)

        zeros = jnp.zeros((_L,), jnp.float32)

        def batch_body(k, carry):
            b = wid * _NBATCH + k
            base = b * _F

            copies = []
            for i in range(_NCH):
                def sub(q, c2, i=i):
                    g = g_v[pl.ds(i * _CH + q * _L, _L)]
                    idx_v[i, pl.ds(q * _L, _L)] = g + base
                    return c2
                lax.fori_loop(0, _CH // _L, sub, 0, unroll=False)
                copies.append(pltpu.async_copy(
                    ll_hbm.at[idx_v.at[i]],
                    buf_v.at[pl.ds(i * _CH, _CH)],
                    gsem))
            for cp in copies:
                cp.wait()

            for g in range(_NBAD // _L):
                bt = pinv_v[pl.ds(_F + g * _L, _L)]
                for col in range(_D):
                    colv = jnp.full((_L,), col, jnp.int32)
                    plsc.store_scatter(buf_v, [bt, colv], zeros)

            pltpu.sync_copy(buf_v, out_hbm.at[pl.ds(b * _T, _T)])
            return carry

        lax.fori_loop(0, _NBATCH, batch_body, 0, unroll=False)

    return packing


_packing = _make_packing_kernel()


def kernel(ll, perm):
    b, f, c, r = ll.shape
    ll_flat = ll.reshape(b * f, c * r)
    out_flat = _packing(ll_flat, perm)
    return out_flat.reshape(b, _T, c, r)


# speedup vs baseline: 1.9047x; 1.9047x over previous
import functools

import jax
import jax.numpy as jnp
from jax import lax
from jax.experimental import pallas as pl
from jax.experimental.pallas import tpu as pltpu
from jax.experimental.pallas import tpu_sc as plsc

_B, _F, _C, _R = 512, 2000, 8, 4
_T = 2048
_L = 16
_NC, _NS = 2, 16
_NW = _NC * _NS

_NIN = _F * _C * 4
_NOUT = _B * _C * 16
_W = 512
_NTT = _T // 128
_HS = 64


def _make_packing_kernel(interpret=False):
    mesh = plsc.VectorSubcoreMesh(
        core_axis_name="c", subcore_axis_name="s",
        num_cores=_NC, num_subcores=_NS)

    @functools.partial(
        pl.kernel,
        out_type=jax.ShapeDtypeStruct((_NOUT, _W), jnp.float32),
        mesh=mesh,
        scratch_types=[
            pltpu.VMEM((_T,), jnp.int32),
            pltpu.VMEM((2, _HS), jnp.int32),
            pltpu.VMEM((1, 128), jnp.int32),
            pltpu.VMEM((_HS + 1, _W), jnp.float32),
            pltpu.VMEM((128, _W), jnp.float32),
            pltpu.SemaphoreType.DMA,
            pltpu.SemaphoreType.DMA,
        ],
        interpret=interpret,
        compiler_params=pltpu.CompilerParams(
            needs_layout_passes=False, use_tc_tiling_on_sc=False),
    )
    def packing(lv_hbm, perm_hbm, ov_hbm,
                perm_v, gidx_v, sidx_v, inb_v, outb_v, gsem, wsem):
        wid = lax.axis_index("s") * _NC + lax.axis_index("c")
        c = wid // 4
        bt = wid % 4
        pltpu.sync_copy(perm_hbm, perm_v)

        lane = lax.iota(jnp.int32, _L)
        zeros = jnp.zeros((_L,), jnp.float32)

        for q in range(_W // _L):
            inb_v[_HS, pl.ds(q * _L, _L)] = zeros

        obase = bt * 128 * _C * 16 + c * 16

        def tt_body(tt, carry):
            for q in range(128 // _L):
                sidx_v[0, pl.ds(q * _L, _L)] = (
                    (lane + q * _L) * (_C * 16) + (obase + tt))

            for s in range(2):
                for q in range(_HS // _L):
                    pv = perm_v[pl.ds(tt * 128 + s * _HS + q * _L, _L)]
                    gidx_v[s, pl.ds(q * _L, _L)] = (
                        jnp.minimum(pv, _F - 1) * (_C * 4) + (c * 4 + bt))
                pltpu.async_copy(
                    lv_hbm.at[gidx_v.at[s]],
                    inb_v.at[pl.ds(0, _HS)], gsem).wait()

                fi = []
                colw = []
                for tlc in range(_HS // _L):
                    pv = perm_v[pl.ds(tt * 128 + s * _HS + tlc * _L, _L)]
                    fi.append(jnp.where(pv < _F, lane + tlc * _L, _HS))
                    colw.append([lane + (r * 128 + s * _HS + tlc * _L)
                                 for r in range(_R)])

                def j_body(j, c2):
                    rowv = jnp.full((_L,), j, jnp.int32)
                    for r in range(_R):
                        colg = jnp.full((_L,), r * 128 + j, jnp.int32)
                        for tlc in range(_HS // _L):
                            v = plsc.load_gather(inb_v, [fi[tlc], colg])
                            plsc.store_scatter(
                                outb_v, [rowv, colw[tlc][r]], v)
                    return c2

                lax.fori_loop(0, 128, j_body, 0, unroll=False)

            pltpu.async_copy(
                outb_v, ov_hbm.at[sidx_v.at[0]], wsem).wait()
            return carry

        lax.fori_loop(0, _NTT, tt_body, 0, unroll=False)

    return packing


_packing = _make_packing_kernel()


def kernel(ll, perm):
    lv = (ll.reshape(4, 128, _F, _C, _R)
            .transpose(2, 3, 0, 4, 1)
            .reshape(_NIN, _W))
    ov = _packing(lv, perm)
    out = (ov.reshape(_B, _C, _NTT, _R, 128)
             .transpose(0, 2, 4, 1, 3)
             .reshape(_B, _T, _C, _R))
    return out


# speedup vs baseline: 3.1700x; 1.6643x over previous
import functools

import jax
import jax.numpy as jnp
from jax import lax
from jax.experimental import pallas as pl
from jax.experimental.pallas import tpu as pltpu
from jax.experimental.pallas import tpu_sc as plsc

_B, _F, _C, _R = 512, 2000, 8, 4
_T = 2048
_L = 16
_NC, _NS = 2, 16
_NW = _NC * _NS

_NIN = _F * _C * 4
_NOUT = _B * _C * 16
_W = 512
_NTT = _T // 128
_HS = 64


def _make_packing_kernel(interpret=False):
    mesh = plsc.VectorSubcoreMesh(
        core_axis_name="c", subcore_axis_name="s",
        num_cores=_NC, num_subcores=_NS)

    @functools.partial(
        pl.kernel,
        out_type=jax.ShapeDtypeStruct((_NOUT, _W), jnp.float32),
        mesh=mesh,
        scratch_types=[
            pltpu.VMEM((_T,), jnp.int32),
            pltpu.VMEM((2, _HS), jnp.int32),
            pltpu.VMEM((1, 128), jnp.int32),
            pltpu.VMEM((_HS + 1, _W), jnp.float32),
            pltpu.VMEM((128, _W), jnp.float32),
            pltpu.SemaphoreType.DMA,
            pltpu.SemaphoreType.DMA,
        ],
        interpret=interpret,
        compiler_params=pltpu.CompilerParams(
            needs_layout_passes=False, use_tc_tiling_on_sc=False),
    )
    def packing(lv_hbm, perm_hbm, ov_hbm,
                perm_v, gidx_v, sidx_v, inb_v, outb_v, gsem, wsem):
        wid = lax.axis_index("s") * _NC + lax.axis_index("c")
        c = wid // 4
        bt = wid % 4
        pltpu.sync_copy(perm_hbm, perm_v)

        lane = lax.iota(jnp.int32, _L)
        zeros = jnp.zeros((_L,), jnp.float32)

        for q in range(_W // _L):
            inb_v[_HS, pl.ds(q * _L, _L)] = zeros

        obase = bt * 128 * _C * 16 + c * 16

        def tt_body(tt, carry):
            for q in range(128 // _L):
                sidx_v[0, pl.ds(q * _L, _L)] = (
                    (lane + q * _L) * (_C * 16) + (obase + tt))

            for s in range(2):
                for q in range(_HS // _L):
                    pv = perm_v[pl.ds(tt * 128 + s * _HS + q * _L, _L)]
                    gidx_v[s, pl.ds(q * _L, _L)] = (
                        jnp.minimum(pv, _F - 1) * (_C * 4) + (c * 4 + bt))
                pltpu.async_copy(
                    lv_hbm.at[gidx_v.at[s]],
                    inb_v.at[pl.ds(0, _HS)], gsem).wait()

                fi = []
                colw = []
                for tlc in range(_HS // _L):
                    pv = perm_v[pl.ds(tt * 128 + s * _HS + tlc * _L, _L)]
                    fi.append(jnp.where(pv < _F, lane + tlc * _L, _HS))
                    colw.append([lane + (r * 128 + s * _HS + tlc * _L)
                                 for r in range(_R)])

                @plsc.parallel_loop(0, 128, unroll=2)
                def j_body(j):
                    rowv = jnp.full((_L,), j, jnp.int32)
                    for r in range(_R):
                        colg = jnp.full((_L,), r * 128 + j, jnp.int32)
                        for tlc in range(_HS // _L):
                            v = plsc.load_gather(inb_v, [fi[tlc], colg])
                            plsc.store_scatter(
                                outb_v, [rowv, colw[tlc][r]], v)

            pltpu.async_copy(
                outb_v, ov_hbm.at[sidx_v.at[0]], wsem).wait()
            return carry

        lax.fori_loop(0, _NTT, tt_body, 0, unroll=False)

    return packing


_packing = _make_packing_kernel()


def kernel(ll, perm):
    lv = (ll.reshape(4, 128, _F, _C, _R)
            .transpose(2, 3, 0, 4, 1)
            .reshape(_NIN, _W))
    ov = _packing(lv, perm)
    out = (ov.reshape(_B, _C, _NTT, _R, 128)
             .transpose(0, 2, 4, 1, 3)
             .reshape(_B, _T, _C, _R))
    return out


# speedup vs baseline: 3.2168x; 1.0148x over previous
import functools

import jax
import jax.numpy as jnp
from jax import lax
from jax.experimental import pallas as pl
from jax.experimental.pallas import tpu as pltpu
from jax.experimental.pallas import tpu_sc as plsc

_B, _F, _C, _R = 512, 2000, 8, 4
_T = 2048
_L = 16
_NC, _NS = 2, 16
_NW = _NC * _NS

_NIN = _F * _C * 4
_NOUT = _B * _C * 16
_W = 512
_NTT = _T // 128
_HS = 64


def _make_packing_kernel(interpret=False):
    mesh = plsc.VectorSubcoreMesh(
        core_axis_name="c", subcore_axis_name="s",
        num_cores=_NC, num_subcores=_NS)

    @functools.partial(
        pl.kernel,
        out_type=jax.ShapeDtypeStruct((_NOUT, _W), jnp.float32),
        mesh=mesh,
        scratch_types=[
            pltpu.VMEM((_T,), jnp.int32),
            pltpu.VMEM((2, _HS), jnp.int32),
            pltpu.VMEM((1, 128), jnp.int32),
            pltpu.VMEM((_HS + 1, _W), jnp.float32),
            pltpu.VMEM((128, _W), jnp.float32),
            pltpu.SemaphoreType.DMA,
            pltpu.SemaphoreType.DMA,
        ],
        interpret=interpret,
        compiler_params=pltpu.CompilerParams(
            needs_layout_passes=False, use_tc_tiling_on_sc=False),
    )
    def packing(lv_hbm, perm_hbm, ov_hbm,
                perm_v, gidx_v, sidx_v, inb_v, outb_v, gsem, wsem):
        wid = lax.axis_index("s") * _NC + lax.axis_index("c")
        c = wid // 4
        bt = wid % 4
        pltpu.sync_copy(perm_hbm, perm_v)

        lane = lax.iota(jnp.int32, _L)
        zeros = jnp.zeros((_L,), jnp.float32)

        for q in range(_W // _L):
            inb_v[_HS, pl.ds(q * _L, _L)] = zeros

        obase = bt * 128 * _C * 16 + c * 16

        def tt_body(tt, carry):
            for q in range(128 // _L):
                sidx_v[0, pl.ds(q * _L, _L)] = (
                    (lane + q * _L) * (_C * 16) + (obase + tt))

            for s in range(2):
                for q in range(_HS // _L):
                    pv = perm_v[pl.ds(tt * 128 + s * _HS + q * _L, _L)]
                    gidx_v[s, pl.ds(q * _L, _L)] = (
                        jnp.minimum(pv, _F - 1) * (_C * 4) + (c * 4 + bt))
                pltpu.async_copy(
                    lv_hbm.at[gidx_v.at[s]],
                    inb_v.at[pl.ds(0, _HS)], gsem).wait()

                fi = []
                colw = []
                for tlc in range(_HS // _L):
                    pv = perm_v[pl.ds(tt * 128 + s * _HS + tlc * _L, _L)]
                    fi.append(jnp.where(pv < _F, lane + tlc * _L, _HS))
                    colw.append([lane + (r * 128 + s * _HS + tlc * _L)
                                 for r in range(_R)])

                @plsc.parallel_loop(0, 128, unroll=4)
                def j_body(j):
                    for r in range(_R):
                        colg = jnp.full((_L,), r * 128 + j, jnp.int32)
                        for tlc in range(_HS // _L):
                            v = plsc.load_gather(inb_v, [fi[tlc], colg])
                            outb_v[j, pl.ds(r * 128 + s * _HS + tlc * _L,
                                            _L)] = v

            pltpu.async_copy(
                outb_v, ov_hbm.at[sidx_v.at[0]], wsem).wait()
            return carry

        lax.fori_loop(0, _NTT, tt_body, 0, unroll=False)

    return packing


_packing = _make_packing_kernel()


def kernel(ll, perm):
    lv = (ll.reshape(4, 128, _F, _C, _R)
            .transpose(2, 3, 0, 4, 1)
            .reshape(_NIN, _W))
    ov = _packing(lv, perm)
    out = (ov.reshape(_B, _C, _NTT, _R, 128)
             .transpose(0, 2, 4, 1, 3)
             .reshape(_B, _T, _C, _R))
    return out
